# baseline (speedup 1.0000x reference)
import functools

import numpy as np
import jax
import jax.numpy as jnp
from jax import lax
from jax.experimental import pallas as pl
from jax.experimental.pallas import tpu as pltpu
from jax.experimental.pallas import tpu_sc as plsc

_BS = 64
_NB = 4096 // _BS
_L = 16


_T16_BITS = (48128, 47529, 47190, 46763, 46329, 45801, 45075, 43363,
             10595, 12307, 13033, 13561, 13995, 14422, 14761, 15360)


def _nf4_tables():
    t16 = np.array(_T16_BITS, dtype=np.uint16).view(np.float16)
    t32 = t16.astype(np.float32)
    u = (t32 + np.float32(1.0)) / np.float32(2.0)
    mids = (t32[:-1] + t32[1:]) * np.float32(0.5)
    c = (mids + np.float32(1.0)) * np.float32(0.5)
    return t32, u, c


_T32, _U, _C = _nf4_tables()

_MAGIC = np.float32(12582912.0)


def _sc_body(x_hbm, o_hbm, xbuf, obuf, tab, bnd, in_sem, out_sem,
             *, rows_per_w, ch, ncols):
    nc = 2
    wid = lax.axis_index("s") * nc + lax.axis_index("c")
    lane = lax.broadcasted_iota(jnp.int32, (_L,), 0)
    gstride = lane * _BS
    sstride = lane * 16
    n_chunks = rows_per_w // ch
    cbytes = ch * ncols
    wrow = wid * rows_per_w

    def start_in(ci, slot):
        for r in range(ch):
            pltpu.async_copy(x_hbm.at[wrow + ci * ch + r],
                             xbuf.at[pl.ds(slot * cbytes + r * ncols, ncols)],
                             in_sem)

    def wait_in(ci, slot):
        for r in range(ch):
            pltpu.make_async_copy(
                x_hbm.at[wrow + ci * ch + r],
                xbuf.at[pl.ds(slot * cbytes + r * ncols, ncols)],
                in_sem).wait()

    def start_out(ci, slot):
        for r in range(ch):
            pltpu.async_copy(obuf.at[pl.ds(slot * cbytes + r * ncols, ncols)],
                             o_hbm.at[wrow + ci * ch + r], out_sem)

    def wait_out(ci, slot):
        for r in range(ch):
            pltpu.make_async_copy(
                obuf.at[pl.ds(slot * cbytes + r * ncols, ncols)],
                o_hbm.at[wrow + ci * ch + r],
                out_sem).wait()

    start_in(0, 0)

    def chunk_body(ci, _c):
        slot = lax.rem(ci, 2)
        sbase = slot * cbytes
        wait_in(ci, slot)

        @pl.when(ci + 1 < n_chunks)
        def _():
            start_in(ci + 1, 1 - slot)

        @pl.when(ci >= 2)
        def _():
            wait_out(ci - 2, slot)

        def row_body(r, _r):
            ro = sbase + r * ncols
            mins = []
            scales = []
            for g in range(_NB // _L):
                gbase = gstride + (ro + g * (_L * _BS))
                lo = hi = None
                for e0 in range(0, _BS, 8):
                    vv = [plsc.load_gather(xbuf, [gbase + (e0 + t)])
                          for t in range(8)]
                    l = jnp.minimum(
                        jnp.minimum(jnp.minimum(vv[0], vv[1]),
                                    jnp.minimum(vv[2], vv[3])),
                        jnp.minimum(jnp.minimum(vv[4], vv[5]),
                                    jnp.minimum(vv[6], vv[7])))
                    h = jnp.maximum(
                        jnp.maximum(jnp.maximum(vv[0], vv[1]),
                                    jnp.maximum(vv[2], vv[3])),
                        jnp.maximum(jnp.maximum(vv[4], vv[5]),
                                    jnp.maximum(vv[6], vv[7])))
                    lo = l if lo is None else jnp.minimum(lo, l)
                    hi = h if hi is None else jnp.maximum(hi, h)
                mins.append(lo)
                scales.append(hi - lo)

            s01 = jnp.minimum(scales[0], scales[1])
            s23 = jnp.minimum(scales[2], scales[3])
            smin = jnp.min(jnp.minimum(s01, s23))
            x01 = jnp.maximum(scales[0], scales[1])
            x23 = jnp.maximum(scales[2], scales[3])
            smax = jnp.max(jnp.maximum(x01, x23))
            ds = smax - smin + jnp.float32(1e-8)
            srange = smax - smin

            for g in range(_NB // _L):
                t = (scales[g] - smin) / ds * jnp.float32(255.0)
                qs = (t + _MAGIC) - _MAGIC
                sdq = smin + qs / jnp.float32(255.0) * srange
                d = scales[g] + jnp.float32(1e-8)
                sb = sstride + (g * _L * 16)
                for i in range(16):
                    plsc.store_scatter(tab, [sb + i],
                                       jnp.float32(_U[i]) * sdq + mins[g])
                for i in range(15):
                    plsc.store_scatter(bnd, [sb + i],
                                       d * jnp.float32(_C[i]) + mins[g])

            @plsc.parallel_loop(0, _NB, unroll=2)
            def b_body(b):
                cnt0 = jnp.full((_L,), 0, jnp.int32) + b * 16
                eb = ro + b * _BS
                for j in range(_BS // _L):
                    v = xbuf[pl.ds(eb + j * _L, _L)]
                    cnt = cnt0
                    for step in (8, 4, 2, 1):
                        m = plsc.load_gather(bnd, [cnt + (step - 1)])
                        cnt = jnp.where(v > m, cnt + step, cnt)
                    obuf[pl.ds(eb + j * _L, _L)] = plsc.load_gather(tab, [cnt])

            return _r

        lax.fori_loop(0, ch, row_body, 0)
        start_out(ci, slot)
        return _c

    lax.fori_loop(0, n_chunks, chunk_body, 0)
    wait_out(n_chunks - 2, lax.rem(n_chunks - 2, 2))
    wait_out(n_chunks - 1, lax.rem(n_chunks - 1, 2))


def _tc_body(x_ref, o_ref, *, rows):
    x = x_ref[...]
    xb = x.reshape(rows, _NB, _BS)
    xmin = jnp.min(xb, axis=2)
    xmax = jnp.max(xb, axis=2)
    scales = xmax - xmin
    d = scales + jnp.float32(1e-8)
    smin = jnp.min(scales, axis=1, keepdims=True)
    smax = jnp.max(scales, axis=1, keepdims=True)
    ds = smax - smin + jnp.float32(1e-8)
    qs = jnp.round((scales - smin) / ds * jnp.float32(255.0))
    scales_d = smin + qs / jnp.float32(255.0) * (smax - smin)
    y = xb - xmin[:, :, None]
    u = jnp.full(xb.shape, _U[0], dtype=jnp.float32)
    for i in range(15):
        thr = d * jnp.float32(_C[i])
        u = jnp.where(y > thr[:, :, None], jnp.float32(_U[i + 1]), u)
    w = u * scales_d[:, :, None] + xmin[:, :, None]
    o_ref[...] = w.reshape(rows, _NB * _BS)


_SC_ROWS = 448


@jax.jit
def kernel(x):
    B, C = x.shape
    nw = 32
    rows_per_w = _SC_ROWS // nw
    ch = 7
    mesh = plsc.VectorSubcoreMesh(core_axis_name="c", subcore_axis_name="s")
    k = functools.partial(
        pl.kernel,
        mesh=mesh,
        compiler_params=pltpu.CompilerParams(
            needs_layout_passes=False, use_tc_tiling_on_sc=True),
        out_type=jax.ShapeDtypeStruct((_SC_ROWS, C), jnp.float32),
        scratch_types=[
            pltpu.VMEM((2 * ch * C,), jnp.float32),
            pltpu.VMEM((2 * ch * C,), jnp.float32),
            pltpu.VMEM((_NB * 16,), jnp.float32),
            pltpu.VMEM((_NB * 16,), jnp.float32),
            pltpu.SemaphoreType.DMA,
            pltpu.SemaphoreType.DMA,
        ],
    )(functools.partial(_sc_body, rows_per_w=rows_per_w, ch=ch, ncols=C))
    sc_out = k(x)

    tc_rows = B - _SC_ROWS
    blk = 8
    tc_out = pl.pallas_call(
        functools.partial(_tc_body, rows=blk),
        grid=(tc_rows // blk,),
        in_specs=[pl.BlockSpec((blk, C), lambda i: (i + _SC_ROWS // blk, 0))],
        out_specs=pl.BlockSpec((blk, C), lambda i: (i, 0)),
        out_shape=jax.ShapeDtypeStruct((tc_rows, C), jnp.float32),
    )(x)
    return jnp.concatenate([sc_out, tc_out], axis=0)


# speedup vs baseline: 1.1288x; 1.1288x over previous
import functools

import numpy as np
import jax
import jax.numpy as jnp
from jax import lax
from jax.experimental import pallas as pl
from jax.experimental.pallas import tpu as pltpu
from jax.experimental.pallas import tpu_sc as plsc

_BS = 64
_NB = 4096 // _BS
_L = 16


_T16_BITS = (48128, 47529, 47190, 46763, 46329, 45801, 45075, 43363,
             10595, 12307, 13033, 13561, 13995, 14422, 14761, 15360)


def _nf4_tables():
    t16 = np.array(_T16_BITS, dtype=np.uint16).view(np.float16)
    t32 = t16.astype(np.float32)
    u = (t32 + np.float32(1.0)) / np.float32(2.0)
    mids = (t32[:-1] + t32[1:]) * np.float32(0.5)
    c = (mids + np.float32(1.0)) * np.float32(0.5)
    return t32, u, c


_T32, _U, _C = _nf4_tables()

_MAGIC = np.float32(12582912.0)


def _sc_body(x_hbm, o_hbm, xbuf, obuf, tab, bnd, in_sem, out_sem,
             *, rows_per_w, ch, ncols):
    nc = 2
    wid = lax.axis_index("s") * nc + lax.axis_index("c")
    lane = lax.broadcasted_iota(jnp.int32, (_L,), 0)
    gstride = lane * _BS
    sstride = lane * 16
    n_chunks = rows_per_w // ch
    cbytes = ch * ncols
    wrow = wid * rows_per_w

    def start_in(ci, slot):
        for r in range(ch):
            pltpu.async_copy(x_hbm.at[wrow + ci * ch + r],
                             xbuf.at[pl.ds(slot * cbytes + r * ncols, ncols)],
                             in_sem)

    def wait_in(ci, slot):
        for r in range(ch):
            pltpu.make_async_copy(
                x_hbm.at[wrow + ci * ch + r],
                xbuf.at[pl.ds(slot * cbytes + r * ncols, ncols)],
                in_sem).wait()

    def start_out(ci, slot):
        for r in range(ch):
            pltpu.async_copy(obuf.at[pl.ds(slot * cbytes + r * ncols, ncols)],
                             o_hbm.at[wrow + ci * ch + r], out_sem)

    def wait_out(ci, slot):
        for r in range(ch):
            pltpu.make_async_copy(
                obuf.at[pl.ds(slot * cbytes + r * ncols, ncols)],
                o_hbm.at[wrow + ci * ch + r],
                out_sem).wait()

    start_in(0, 0)

    def chunk_body(ci, _c):
        slot = lax.rem(ci, 2)
        sbase = slot * cbytes
        wait_in(ci, slot)

        @pl.when(ci + 1 < n_chunks)
        def _():
            start_in(ci + 1, 1 - slot)

        @pl.when(ci >= 2)
        def _():
            wait_out(ci - 2, slot)

        def row_body(r, _r):
            ro = sbase + r * ncols
            mins = []
            scales = []
            for g in range(_NB // _L):
                gbase = gstride + (ro + g * (_L * _BS))
                lo = hi = None
                for e0 in range(0, _BS, 8):
                    vv = [plsc.load_gather(xbuf, [gbase + (e0 + t)])
                          for t in range(8)]
                    l = jnp.minimum(
                        jnp.minimum(jnp.minimum(vv[0], vv[1]),
                                    jnp.minimum(vv[2], vv[3])),
                        jnp.minimum(jnp.minimum(vv[4], vv[5]),
                                    jnp.minimum(vv[6], vv[7])))
                    h = jnp.maximum(
                        jnp.maximum(jnp.maximum(vv[0], vv[1]),
                                    jnp.maximum(vv[2], vv[3])),
                        jnp.maximum(jnp.maximum(vv[4], vv[5]),
                                    jnp.maximum(vv[6], vv[7])))
                    lo = l if lo is None else jnp.minimum(lo, l)
                    hi = h if hi is None else jnp.maximum(hi, h)
                mins.append(lo)
                scales.append(hi - lo)

            s01 = jnp.minimum(scales[0], scales[1])
            s23 = jnp.minimum(scales[2], scales[3])
            smin = jnp.min(jnp.minimum(s01, s23))
            x01 = jnp.maximum(scales[0], scales[1])
            x23 = jnp.maximum(scales[2], scales[3])
            smax = jnp.max(jnp.maximum(x01, x23))
            ds = smax - smin + jnp.float32(1e-8)
            srange = smax - smin

            for g in range(_NB // _L):
                t = (scales[g] - smin) / ds * jnp.float32(255.0)
                qs = (t + _MAGIC) - _MAGIC
                sdq = smin + qs / jnp.float32(255.0) * srange
                d = scales[g] + jnp.float32(1e-8)
                sb = sstride + (g * _L * 16)
                for i in range(16):
                    plsc.store_scatter(tab, [sb + i],
                                       jnp.float32(_U[i]) * sdq + mins[g])
                for i in range(15):
                    plsc.store_scatter(bnd, [sb + i],
                                       d * jnp.float32(_C[i]) + mins[g])

            @plsc.parallel_loop(0, _NB, unroll=2)
            def b_body(b):
                cnt0 = jnp.full((_L,), 0, jnp.int32) + b * 16
                eb = ro + b * _BS
                for j in range(_BS // _L):
                    v = xbuf[pl.ds(eb + j * _L, _L)]
                    cnt = cnt0
                    for step in (8, 4, 2, 1):
                        m = plsc.load_gather(bnd, [cnt + (step - 1)])
                        cnt = jnp.where(v > m, cnt + step, cnt)
                    obuf[pl.ds(eb + j * _L, _L)] = plsc.load_gather(tab, [cnt])

            return _r

        lax.fori_loop(0, ch, row_body, 0)
        start_out(ci, slot)
        return _c

    lax.fori_loop(0, n_chunks, chunk_body, 0)
    wait_out(n_chunks - 2, lax.rem(n_chunks - 2, 2))
    wait_out(n_chunks - 1, lax.rem(n_chunks - 1, 2))


def _tc_body(x_ref, o_ref, *, rows):
    x = x_ref[...]
    xb = x.reshape(rows, _NB, _BS)
    xmin = jnp.min(xb, axis=2)
    xmax = jnp.max(xb, axis=2)
    scales = xmax - xmin
    d = scales + jnp.float32(1e-8)
    smin = jnp.min(scales, axis=1, keepdims=True)
    smax = jnp.max(scales, axis=1, keepdims=True)
    ds = smax - smin + jnp.float32(1e-8)
    qs = jnp.round((scales - smin) / ds * jnp.float32(255.0))
    scales_d = smin + qs / jnp.float32(255.0) * (smax - smin)
    y = xb - xmin[:, :, None]
    u = jnp.full(xb.shape, _U[0], dtype=jnp.float32)
    for i in range(15):
        thr = d * jnp.float32(_C[i])
        u = jnp.where(y > thr[:, :, None], jnp.float32(_U[i + 1]), u)
    w = u * scales_d[:, :, None] + xmin[:, :, None]
    o_ref[...] = w.reshape(rows, _NB * _BS)


_SC_ROWS = 576


@jax.jit
def kernel(x):
    B, C = x.shape
    nw = 32
    rows_per_w = _SC_ROWS // nw
    ch = 6
    mesh = plsc.VectorSubcoreMesh(core_axis_name="c", subcore_axis_name="s")
    k = functools.partial(
        pl.kernel,
        mesh=mesh,
        compiler_params=pltpu.CompilerParams(
            needs_layout_passes=False, use_tc_tiling_on_sc=True),
        out_type=jax.ShapeDtypeStruct((_SC_ROWS, C), jnp.float32),
        scratch_types=[
            pltpu.VMEM((2 * ch * C,), jnp.float32),
            pltpu.VMEM((2 * ch * C,), jnp.float32),
            pltpu.VMEM((_NB * 16,), jnp.float32),
            pltpu.VMEM((_NB * 16,), jnp.float32),
            pltpu.SemaphoreType.DMA,
            pltpu.SemaphoreType.DMA,
        ],
    )(functools.partial(_sc_body, rows_per_w=rows_per_w, ch=ch, ncols=C))
    sc_out = k(x)

    tc_rows = B - _SC_ROWS
    blk = 8
    tc_out = pl.pallas_call(
        functools.partial(_tc_body, rows=blk),
        grid=(tc_rows // blk,),
        in_specs=[pl.BlockSpec((blk, C), lambda i: (i + _SC_ROWS // blk, 0))],
        out_specs=pl.BlockSpec((blk, C), lambda i: (i, 0)),
        out_shape=jax.ShapeDtypeStruct((tc_rows, C), jnp.float32),
    )(x)
    return jnp.concatenate([sc_out, tc_out], axis=0)
